# Initial kernel scaffold; baseline (speedup 1.0000x reference)
import jax
import jax.numpy as jnp
from jax.experimental import pallas as pl


def kernel(x, params):
    raise NotImplementedError("write your pallas kernel here")


# speedup vs baseline: 4.2272x; 4.2272x over previous
import functools

import jax
import jax.numpy as jnp
import numpy as np
from jax import lax
import jax.experimental.pallas as pl
from jax.experimental.pallas import tpu as pltpu

_DIM = 96
_NE = 5
_IMG = 64
_HW = _IMG * _IMG
_INV_SQRT2 = np.float32(0.7071067811865476)


def _gelu(v):
    return 0.5 * v * (1.0 + lax.erf(v * _INV_SQRT2))


def _router_kernel(xt_ref, w1t_ref, b1_ref, w2t_ref, b2_ref, w_ref):
    nb = xt_ref.shape[0]
    pooled = jnp.concatenate(
        [jnp.mean(xt_ref[b], axis=0, keepdims=True) for b in range(nb)], axis=0
    )
    h = _gelu(jnp.dot(pooled, w1t_ref[...], preferred_element_type=jnp.float32)
              + b1_ref[...])
    logits = (jnp.dot(h, w2t_ref[...], preferred_element_type=jnp.float32)
              + b2_ref[...])
    cols = []
    for e in range(_NE):
        ce = logits[:, e:e + 1]
        rank = jnp.sum(jnp.where(logits > ce, 1.0, 0.0), axis=1, keepdims=True)
        for j in range(e):
            rank = rank + jnp.where(logits[:, j:j + 1] == ce, 1.0, 0.0)
        cols.append(rank)
    sel = jnp.concatenate(cols, axis=1) < 2.5
    lm = jnp.where(sel, logits, jnp.float32(-1e30))
    m = jnp.max(lm, axis=1, keepdims=True)
    ex = jnp.where(sel, jnp.exp(logits - m), 0.0)
    w_ref[...] = ex / jnp.sum(ex, axis=1, keepdims=True)


def _moe_kernel(treedef, w_ref, xt_ref, *args):
    out_ref = args[-1]
    P = jax.tree_util.tree_unflatten(treedef, args[:-1])
    b = pl.program_id(0)
    x = xt_ref[0]

    row = lax.broadcasted_iota(jnp.int32, (_HW, 1), 0)
    wcol = lax.bitwise_and(row, _IMG - 1)

    def shift(v, dh, dw):
        s = dh * _IMG + dw
        r = v if s == 0 else jnp.roll(v, -s, axis=0)
        conds = []
        if dh > 0:
            conds.append(row < _HW - _IMG * dh)
        if dh < 0:
            conds.append(row >= -_IMG * dh)
        if dw > 0:
            conds.append(wcol < _IMG - dw)
        if dw < 0:
            conds.append(wcol >= -dw)
        if not conds:
            return r
        m = conds[0]
        for c in conds[1:]:
            m = m & c
        return jnp.where(m, r, 0.0)

    def mm(a, wt, bias):
        return jnp.dot(a, wt[...], preferred_element_type=jnp.float32) + bias[...]

    def ln_lanes(v, g_ref, be_ref):
        mu = jnp.mean(v, axis=1, keepdims=True)
        var = jnp.mean((v - mu) * (v - mu), axis=1, keepdims=True)
        return (v - mu) * lax.rsqrt(var + 1e-6) * g_ref[...] + be_ref[...]

    def att_fuse(feats, q):
        pooled = jnp.mean(feats, axis=0, keepdims=True)
        a = _gelu(mm(pooled, q['aW1T'], q['ab1']))
        a = jax.nn.sigmoid(mm(a, q['aW2T'], q['ab2']))
        f2 = feats * a
        g = mm(f2, q['fWT'], q['fb'])
        return _gelu(ln_lanes(g, q['fg'], q['fbe']))

    def branches4(ts, q):
        acc = q['bb'][...]
        for k in range(4):
            acc = acc + jnp.dot(ts[k], q['bW' + str(k)][...],
                                preferred_element_type=jnp.float32)
        return _gelu(acc)

    def attn_expert():
        q = P['attn']
        return x + _gelu(mm(x, q['WT'], q['b']))

    def edge_expert():
        q = P['edge']
        n = {(dh, dw): shift(x, dh, dw)
             for dh in (-1, 0, 1) for dw in (-1, 0, 1) if (dh, dw) != (0, 0)}
        sh = ((n[(-1, 1)] - n[(-1, -1)]) + 2.0 * (n[(0, 1)] - n[(0, -1)])
              + (n[(1, 1)] - n[(1, -1)]))
        sv = ((n[(1, -1)] + 2.0 * n[(1, 0)] + n[(1, 1)])
              - (n[(-1, -1)] + 2.0 * n[(-1, 0)] + n[(-1, 1)]))
        lapv = n[(-1, 0)] + n[(0, -1)] + n[(0, 1)] + n[(1, 0)] - 4.0 * x
        d1 = n[(-1, -1)] - n[(-1, 1)] - n[(1, -1)] + n[(1, 1)]
        sobel = jnp.sqrt(sh * sh + sv * sv + 1e-08)
        lapE = jnp.abs(lapv)
        diag = jnp.abs(d1)
        gmag = jnp.sqrt(sobel * sobel + lapE * lapE + 1e-08)
        feats = branches4((sobel, lapE, diag, gmag), q)
        return att_fuse(feats, q) + x

    def freq_expert():
        q = P['freq']
        s8 = None
        for dh in (-1, 0, 1):
            for dw in (-1, 0, 1):
                if (dh, dw) == (0, 0):
                    continue
                t = shift(x, dh, dw)
                s8 = t if s8 is None else s8 + t
        souter = None
        for dh in (-2, -1, 0, 1, 2):
            for dw in (-2, -1, 0, 1, 2):
                if max(abs(dh), abs(dw)) != 2:
                    continue
                t = shift(x, dh, dw)
                souter = t if souter is None else souter + t
        low = (x + s8) * np.float32(1.0 / 9.0)
        avg5 = (x + s8 + souter) * np.float32(1.0 / 25.0)
        mid = low - avg5
        high = x - low
        feats = branches4((low, mid, high, x), q)
        return att_fuse(feats, q) + x

    def hybrid_expert():
        q = P['hybrid']
        acc = None
        for i in range(5):
            for j in range(5):
                t = q['dw'][i * 5 + j:i * 5 + j + 1, :] * shift(x, i - 2, j - 2)
                acc = t if acc is None else acc + t
        h = ln_lanes(acc, q['ln_g'], q['ln_b'])
        return x + _gelu(mm(h, q['pwWT'], q['pwb']))

    def texture_expert():
        q = P['texture']
        acc = None
        for i in range(3):
            for j in range(3):
                t = q['dw'][i * 3 + j:i * 3 + j + 1, :] * shift(x, i - 1, j - 1)
                acc = t if acc is None else acc + t
        return x + mm(_gelu(acc), q['pwWT'], q['pwb'])

    out_ref[0] = jnp.zeros((_HW, _DIM), jnp.float32)

    def gate(e, fn):
        w = w_ref[b, e]

        @pl.when(w > 0.0)
        def _():
            out_ref[0] += w * fn()

    gate(0, attn_expert)
    gate(1, edge_expert)
    gate(2, hybrid_expert)
    gate(3, freq_expert)
    gate(4, texture_expert)


def _pack_params(params):
    def fold_branch(bp):
        s = bp['g'] * np.float32(1.0 / np.sqrt(1.0 + 1e-5))
        wt = (bp['W'] * s[:, None]).T
        bias = bp['b'] * s + bp['be']
        return wt, bias

    def pack_cf(p):
        d4 = _DIM // 4
        q = {}
        bb = jnp.zeros((1, _DIM), jnp.float32)
        for k, name in enumerate(('b0', 'b1', 'b2', 'b3')):
            wt, bias = fold_branch(p[name])
            q['bW' + str(k)] = (jnp.zeros((_DIM, _DIM), jnp.float32)
                                .at[:, k * d4:(k + 1) * d4].set(wt))
            bb = bb.at[0, k * d4:(k + 1) * d4].set(bias)
        q['bb'] = bb
        q['aW1T'] = p['att_W1'].T
        q['ab1'] = p['att_b1'][None, :]
        q['aW2T'] = p['att_W2'].T
        q['ab2'] = p['att_b2'][None, :]
        q['fWT'] = p['fus_W'].T
        q['fb'] = p['fus_b'][None, :]
        q['fg'] = p['fus_g'][None, :]
        q['fbe'] = p['fus_be'][None, :]
        return q

    return {
        'attn': {'WT': params['attn']['W'].T, 'b': params['attn']['b'][None, :]},
        'edge': pack_cf(params['edge']),
        'freq': pack_cf(params['freq']),
        'hybrid': {
            'dw': params['hybrid']['dw'][:, 0].reshape(_DIM, 25).T,
            'ln_g': params['hybrid']['ln_g'][None, :],
            'ln_b': params['hybrid']['ln_b'][None, :],
            'pwWT': params['hybrid']['pw_W'].T,
            'pwb': params['hybrid']['pw_b'][None, :],
        },
        'texture': {
            'dw': params['texture']['dw'][:, 0].reshape(_DIM, 9).T,
            'pwWT': params['texture']['pw_W'].T,
            'pwb': params['texture']['pw_b'][None, :],
        },
    }


@jax.jit
def kernel(x, params):
    B, C, Hh, Ww = x.shape
    xt = x.reshape(B, C, Hh * Ww).transpose(0, 2, 1)

    r = params['router']
    weights = pl.pallas_call(
        _router_kernel,
        out_shape=jax.ShapeDtypeStruct((B, _NE), jnp.float32),
    )(xt, r['g_W1'].T, r['g_b1'][None, :], r['g_W2'].T, r['g_b2'][None, :])

    packed = _pack_params(params)
    leaves, treedef = jax.tree_util.tree_flatten(packed)

    def full_spec(a):
        nd = a.ndim
        return pl.BlockSpec(a.shape, lambda bi, _n=nd: (0,) * _n)

    out = pl.pallas_call(
        functools.partial(_moe_kernel, treedef),
        grid=(B,),
        in_specs=[pl.BlockSpec(memory_space=pltpu.SMEM),
                  pl.BlockSpec((1, _HW, C), lambda bi: (bi, 0, 0))]
                 + [full_spec(a) for a in leaves],
        out_specs=pl.BlockSpec((1, _HW, C), lambda bi: (bi, 0, 0)),
        out_shape=jax.ShapeDtypeStruct((B, _HW, C), jnp.float32),
    )(weights, xt, *leaves)
    return out.transpose(0, 2, 1).reshape(B, C, Hh, Ww)


# speedup vs baseline: 4.6967x; 1.1111x over previous
import functools

import jax
import jax.numpy as jnp
import numpy as np
from jax import lax
import jax.experimental.pallas as pl
from jax.experimental.pallas import tpu as pltpu

_DIM = 96
_NE = 5
_IMG = 64
_HW = _IMG * _IMG
_PAD = 2 * _IMG
_SCR = _HW + 2 * _PAD
_INV_SQRT2 = np.float32(0.7071067811865476)


def _gelu(v):
    return 0.5 * v * (1.0 + lax.erf(v * _INV_SQRT2))


def _router_kernel(xt_ref, w1t_ref, b1_ref, w2t_ref, b2_ref, w_ref):
    nb = xt_ref.shape[0]
    pooled = jnp.concatenate(
        [jnp.mean(xt_ref[b], axis=0, keepdims=True) for b in range(nb)], axis=0
    )
    h = _gelu(jnp.dot(pooled, w1t_ref[...], preferred_element_type=jnp.float32)
              + b1_ref[...])
    logits = (jnp.dot(h, w2t_ref[...], preferred_element_type=jnp.float32)
              + b2_ref[...])
    cols = []
    for e in range(_NE):
        ce = logits[:, e:e + 1]
        rank = jnp.sum(jnp.where(logits > ce, 1.0, 0.0), axis=1, keepdims=True)
        for j in range(e):
            rank = rank + jnp.where(logits[:, j:j + 1] == ce, 1.0, 0.0)
        cols.append(rank)
    sel = jnp.concatenate(cols, axis=1) < 2.5
    lm = jnp.where(sel, logits, jnp.float32(-1e30))
    m = jnp.max(lm, axis=1, keepdims=True)
    ex = jnp.where(sel, jnp.exp(logits - m), 0.0)
    w_ref[...] = ex / jnp.sum(ex, axis=1, keepdims=True)


def _moe_kernel(layout, w_ref, xt_ref, pm_ref, out_ref, *scr):
    b = pl.program_id(0)
    x = xt_ref[0]

    def q(name):
        off, nr = layout[name]
        return pm_ref[off:off + nr, :]

    row = lax.broadcasted_iota(jnp.int32, (_HW, 1), 0)
    wcol = lax.bitwise_and(row, _IMG - 1)
    zpad = jnp.zeros((_PAD, _DIM), jnp.float32)
    for dw in (-2, -1, 0, 1, 2):
        sref = scr[dw + 2]
        sref[0:_PAD, :] = zpad
        sref[_PAD + _HW:_SCR, :] = zpad
        if dw == 0:
            sref[_PAD:_PAD + _HW, :] = x
        else:
            r = jnp.roll(x, -dw, axis=0)
            m = (wcol < _IMG - dw) if dw > 0 else (wcol >= -dw)
            sref[_PAD:_PAD + _HW, :] = jnp.where(m, r, 0.0)

    def tap(dh, dw):
        base = _PAD + _IMG * dh
        return scr[dw + 2][base:base + _HW, :]

    def mm(a, wt, bias):
        return jnp.dot(a, wt, preferred_element_type=jnp.float32) + bias

    def ln_lanes(v, g, be):
        mu = jnp.mean(v, axis=1, keepdims=True)
        var = jnp.mean((v - mu) * (v - mu), axis=1, keepdims=True)
        return (v - mu) * lax.rsqrt(var + 1e-6) * g + be

    def att_fuse(feats, pre):
        pooled = jnp.mean(feats, axis=0, keepdims=True)
        a = _gelu(mm(pooled, q(pre + 'aW1T'), q(pre + 'ab1')))
        a = jax.nn.sigmoid(mm(a, q(pre + 'aW2T'), q(pre + 'ab2')))
        g = mm(feats * a, q(pre + 'fWT'), q(pre + 'fb'))
        return _gelu(ln_lanes(g, q(pre + 'fg'), q(pre + 'fbe')))

    def branches4(ts, pre):
        acc = q(pre + 'bb')
        for k in range(4):
            acc = acc + jnp.dot(ts[k], q(pre + 'bW' + str(k)),
                                preferred_element_type=jnp.float32)
        return _gelu(acc)

    def attn_expert():
        return x + _gelu(mm(x, q('attn.WT'), q('attn.b')))

    def edge_expert():
        sh = ((tap(-1, 1) - tap(-1, -1)) + 2.0 * (tap(0, 1) - tap(0, -1))
              + (tap(1, 1) - tap(1, -1)))
        sv = ((tap(1, -1) + 2.0 * tap(1, 0) + tap(1, 1))
              - (tap(-1, -1) + 2.0 * tap(-1, 0) + tap(-1, 1)))
        lapv = tap(-1, 0) + tap(0, -1) + tap(0, 1) + tap(1, 0) - 4.0 * x
        d1 = tap(-1, -1) - tap(-1, 1) - tap(1, -1) + tap(1, 1)
        sobel = jnp.sqrt(sh * sh + sv * sv + 1e-08)
        lapE = jnp.abs(lapv)
        diag = jnp.abs(d1)
        gmag = jnp.sqrt(sobel * sobel + lapE * lapE + 1e-08)
        feats = branches4((sobel, lapE, diag, gmag), 'edge.')
        return att_fuse(feats, 'edge.') + x

    def freq_expert():
        s8 = None
        for dh in (-1, 0, 1):
            for dw in (-1, 0, 1):
                if (dh, dw) == (0, 0):
                    continue
                t = tap(dh, dw)
                s8 = t if s8 is None else s8 + t
        souter = None
        for dh in (-2, -1, 0, 1, 2):
            for dw in (-2, -1, 0, 1, 2):
                if max(abs(dh), abs(dw)) != 2:
                    continue
                t = tap(dh, dw)
                souter = t if souter is None else souter + t
        low = (x + s8) * np.float32(1.0 / 9.0)
        avg5 = (x + s8 + souter) * np.float32(1.0 / 25.0)
        mid = low - avg5
        high = x - low
        feats = branches4((low, mid, high, x), 'freq.')
        return att_fuse(feats, 'freq.') + x

    def hybrid_expert():
        doff = layout['hybrid.dw'][0]
        acc = None
        for i in range(5):
            for j in range(5):
                t = pm_ref[doff + i * 5 + j:doff + i * 5 + j + 1, :] \
                    * tap(i - 2, j - 2)
                acc = t if acc is None else acc + t
        h = ln_lanes(acc, q('hybrid.ln_g'), q('hybrid.ln_b'))
        return x + _gelu(mm(h, q('hybrid.pwWT'), q('hybrid.pwb')))

    def texture_expert():
        doff = layout['texture.dw'][0]
        acc = None
        for i in range(3):
            for j in range(3):
                t = pm_ref[doff + i * 3 + j:doff + i * 3 + j + 1, :] \
                    * tap(i - 1, j - 1)
                acc = t if acc is None else acc + t
        return x + mm(_gelu(acc), q('texture.pwWT'), q('texture.pwb'))

    out_ref[0] = jnp.zeros((_HW, _DIM), jnp.float32)

    def gate(e, fn):
        w = w_ref[b, e]

        @pl.when(w > 0.0)
        def _():
            out_ref[0] += w * fn()

    gate(0, attn_expert)
    gate(1, edge_expert)
    gate(2, hybrid_expert)
    gate(3, freq_expert)
    gate(4, texture_expert)


def _pack_params(params):
    blocks = []
    layout = {}
    cur = [0]

    def add(name, arr):
        arr = jnp.asarray(arr, jnp.float32)
        nr, nc = arr.shape
        if nc < _DIM:
            arr = jnp.pad(arr, ((0, 0), (0, _DIM - nc)))
        layout[name] = (cur[0], nr)
        nr8 = (nr + 7) // 8 * 8
        if nr8 > nr:
            arr = jnp.pad(arr, ((0, nr8 - nr), (0, 0)))
        blocks.append(arr)
        cur[0] += nr8

    def fold_branch(bp):
        s = bp['g'] * np.float32(1.0 / np.sqrt(1.0 + 1e-5))
        return (bp['W'] * s[:, None]).T, bp['b'] * s + bp['be']

    def pack_cf(pre, p):
        d4 = _DIM // 4
        bb = jnp.zeros((1, _DIM), jnp.float32)
        for k, name in enumerate(('b0', 'b1', 'b2', 'b3')):
            wt, bias = fold_branch(p[name])
            add(pre + 'bW' + str(k),
                jnp.zeros((_DIM, _DIM), jnp.float32)
                .at[:, k * d4:(k + 1) * d4].set(wt))
            bb = bb.at[0, k * d4:(k + 1) * d4].set(bias)
        add(pre + 'bb', bb)
        add(pre + 'aW1T', p['att_W1'].T)
        add(pre + 'ab1', p['att_b1'][None, :])
        aw2t = p['att_W2'].T
        add(pre + 'aW2T', jnp.pad(aw2t, ((0, _DIM - aw2t.shape[0]), (0, 0))))
        add(pre + 'ab2', p['att_b2'][None, :])
        add(pre + 'fWT', p['fus_W'].T)
        add(pre + 'fb', p['fus_b'][None, :])
        add(pre + 'fg', p['fus_g'][None, :])
        add(pre + 'fbe', p['fus_be'][None, :])

    add('attn.WT', params['attn']['W'].T)
    add('attn.b', params['attn']['b'][None, :])
    pack_cf('edge.', params['edge'])
    pack_cf('freq.', params['freq'])
    add('hybrid.dw', params['hybrid']['dw'][:, 0].reshape(_DIM, 25).T)
    add('hybrid.ln_g', params['hybrid']['ln_g'][None, :])
    add('hybrid.ln_b', params['hybrid']['ln_b'][None, :])
    add('hybrid.pwWT', params['hybrid']['pw_W'].T)
    add('hybrid.pwb', params['hybrid']['pw_b'][None, :])
    add('texture.dw', params['texture']['dw'][:, 0].reshape(_DIM, 9).T)
    add('texture.pwWT', params['texture']['pw_W'].T)
    add('texture.pwb', params['texture']['pw_b'][None, :])
    return jnp.concatenate(blocks, axis=0), layout


@jax.jit
def kernel(x, params):
    B, C, Hh, Ww = x.shape
    xt = x.reshape(B, C, Hh * Ww).transpose(0, 2, 1)

    r = params['router']
    weights = pl.pallas_call(
        _router_kernel,
        out_shape=jax.ShapeDtypeStruct((B, _NE), jnp.float32),
    )(xt, r['g_W1'].T, r['g_b1'][None, :], r['g_W2'].T, r['g_b2'][None, :])

    pm, layout = _pack_params(params)
    layout = {k: v for k, v in layout.items()}

    out = pl.pallas_call(
        functools.partial(_moe_kernel, layout),
        grid=(B,),
        in_specs=[pl.BlockSpec(memory_space=pltpu.SMEM),
                  pl.BlockSpec((1, _HW, C), lambda bi: (bi, 0, 0)),
                  pl.BlockSpec(pm.shape, lambda bi: (0, 0))],
        out_specs=pl.BlockSpec((1, _HW, C), lambda bi: (bi, 0, 0)),
        out_shape=jax.ShapeDtypeStruct((B, _HW, C), jnp.float32),
        scratch_shapes=[pltpu.VMEM((_SCR, _DIM), jnp.float32)
                        for _ in range(5)],
    )(weights, xt, pm)
    return out.transpose(0, 2, 1).reshape(B, C, Hh, Ww)
